# Initial kernel scaffold; baseline (speedup 1.0000x reference)
import jax
import jax.numpy as jnp
from jax.experimental import pallas as pl


def kernel(xyz, params):
    raise NotImplementedError("write your pallas kernel here")


# speedup vs baseline: 1.0004x; 1.0004x over previous
import jax
import jax.numpy as jnp
from jax.experimental import pallas as pl


def _gather_nb(points, idx):
    return jax.vmap(lambda p, i: p[i])(points, idx)


def _mcg_layer(xyz, feats, p, npoint, nsample):
    B, N, _ = xyz.shape
    if npoint is None:
        centers = jnp.mean(xyz, axis=1, keepdims=True)
        grouped_xyz = xyz[:, None, :, :]
        grouped_feats = feats[:, None, :, :]
    else:
        stride = N // npoint
        centers = xyz[:, ::stride, :][:, :npoint, :]
        d2 = jnp.sum((centers[:, :, None, :] - xyz[:, None, :, :]) ** 2, axis=-1)
        _, idx = jax.lax.top_k(-d2, nsample)
        grouped_xyz = _gather_nb(xyz, idx)
        grouped_feats = _gather_nb(feats, idx)
    rel = grouped_xyz - centers[:, :, None, :]
    dist = jnp.sqrt(jnp.sum(rel ** 2, axis=-1, keepdims=True) + 1e-8)
    ctr = jnp.broadcast_to(centers[:, :, None, :], rel.shape)
    geo = jnp.concatenate([rel, ctr, dist], axis=-1)
    h = jax.nn.relu(geo @ p['W1a'] + p['b1a'])
    w = h @ p['W1b'] + p['b1b']
    lw = jax.nn.relu(w @ p['W2a'] + p['b2a'])
    lw = jax.nn.relu(lw @ p['W2b'] + p['b2b'])
    proj = grouped_feats @ p['Wproj']
    f = jnp.max(proj * lw, axis=2)
    return f, centers


def _head_body(x_ref, w1_ref, a1_ref, w2_ref, a2_ref, w3_ref, b3_ref, o_ref):
    es = 1.0 / jnp.sqrt(1.0 + 1e-5)
    x = x_ref[...]
    x = jnp.maximum(jnp.dot(x, w1_ref[...], preferred_element_type=jnp.float32) * es
                    * a1_ref[0:1, :] + a1_ref[1:2, :], 0.0)
    x = jnp.maximum(jnp.dot(x, w2_ref[...], preferred_element_type=jnp.float32) * es
                    * a2_ref[0:1, :] + a2_ref[1:2, :], 0.0)
    o_ref[...] = jnp.dot(x, w3_ref[...], preferred_element_type=jnp.float32) + b3_ref[0:1, :]


def _head(x, h):
    B = x.shape[0]
    es = 1.0 / jnp.sqrt(1.0 + 1e-5)
    a1 = jnp.stack([h['bn1g'], h['fc1b'] * es * h['bn1g'] + h['bn1b']], axis=0)
    a2 = jnp.stack([h['bn2g'], h['fc2b'] * es * h['bn2g'] + h['bn2b']], axis=0)
    out = pl.pallas_call(
        _head_body,
        out_shape=jax.ShapeDtypeStruct((B, 40), jnp.float32),
    )(x, h['fc1W'], a1, h['fc2W'], a2, h['fc3W'], h['fc3b'][None, :])
    return out


def kernel(xyz, params):
    B = xyz.shape[0]
    pts = jnp.transpose(xyz[:, :3, :], (0, 2, 1))
    norm = jnp.transpose(xyz[:, 3:, :], (0, 2, 1))
    f1, s1 = _mcg_layer(pts, norm, params['layer0'], 512, 32)
    f2, s2 = _mcg_layer(s1, f1, params['layer1'], 256, 32)
    f3, s3 = _mcg_layer(s2, f2, params['layer2'], 128, 32)
    f4, s4 = _mcg_layer(s3, f3, params['layer3'], 32, 32)
    f5, s5 = _mcg_layer(s4, f4, params['layer4'], None, 32)
    x = f5.reshape(B, 1024)
    return (_head(x, params['head']), s3)


# speedup vs baseline: 5.9933x; 5.9911x over previous
import functools

import jax
import jax.numpy as jnp
from jax import lax
from jax.experimental import pallas as pl
from jax.experimental.pallas import tpu as pltpu
from jax.experimental.pallas import tpu_sc as plsc


def _knn_body(xt_ref, xr_ref, c_ref, idx_ref, dist_ref, rel_ref, *rest,
              n, nsample, nf):
    d2_ref = rest[-1]
    feat_ref = rest[0] if nf > 3 else None
    b = pl.program_id(0)
    p = c_ref.shape[1]
    c = c_ref[0]
    d2 = (xt_ref[0, 0:1, :] - c[:, 0:1]) ** 2
    d2 = d2 + (xt_ref[0, 1:2, :] - c[:, 1:2]) ** 2
    d2 = d2 + (xt_ref[0, 2:3, :] - c[:, 2:3]) ** 2
    d2_ref[...] = d2
    iota = lax.broadcasted_iota(jnp.int32, (p, n), 1)
    base = b * n
    xr = xr_ref[0]
    icols, dcols, rcols, fcols = [], [], [], []
    for _ in range(nsample):
        d2 = d2_ref[...]
        m = jnp.min(d2, axis=1, keepdims=True)
        amin = jnp.min(jnp.where(d2 == m, iota, n), axis=1, keepdims=True)
        icols.append(amin + base)
        dcols.append(jnp.sqrt(m + 1e-8))
        eqf = (iota == amin).astype(jnp.float32)
        sel = jnp.dot(eqf, xr, preferred_element_type=jnp.float32)
        rcols.append(sel[:, 0:3] - c)
        if nf > 3:
            fcols.append(sel[:, 3:nf])
        d2_ref[...] = jnp.where(iota == amin, jnp.inf, d2)
    idx_ref[0] = jnp.concatenate(icols, axis=1)
    dist_ref[0] = jnp.concatenate(dcols, axis=1)
    rel_ref[0] = jnp.concatenate(rcols, axis=1)
    if nf > 3:
        feat_ref[0] = jnp.concatenate(fcols, axis=1)


def _knn(xt, xr, centers, nsample):
    batch, _, n = xt.shape
    nf = xr.shape[2]
    npoint = centers.shape[1]
    p = min(64, npoint)
    grid = (batch, npoint // p)
    out_shape = [
        jax.ShapeDtypeStruct((batch, npoint, nsample), jnp.int32),
        jax.ShapeDtypeStruct((batch, npoint, nsample), jnp.float32),
        jax.ShapeDtypeStruct((batch, npoint, nsample * 3), jnp.float32),
    ]
    out_specs = [
        pl.BlockSpec((1, p, nsample), lambda b, j: (b, j, 0)),
        pl.BlockSpec((1, p, nsample), lambda b, j: (b, j, 0)),
        pl.BlockSpec((1, p, nsample * 3), lambda b, j: (b, j, 0)),
    ]
    if nf > 3:
        out_shape.append(
            jax.ShapeDtypeStruct((batch, npoint, nsample * (nf - 3)), jnp.float32))
        out_specs.append(
            pl.BlockSpec((1, p, nsample * (nf - 3)), lambda b, j: (b, j, 0)))
    return pl.pallas_call(
        functools.partial(_knn_body, n=n, nsample=nsample, nf=nf),
        grid=grid,
        in_specs=[
            pl.BlockSpec((1, 3, n), lambda b, j: (b, 0, 0)),
            pl.BlockSpec((1, n, nf), lambda b, j: (b, 0, 0)),
            pl.BlockSpec((1, p, 3), lambda b, j: (b, j, 0)),
        ],
        out_specs=out_specs,
        out_shape=out_shape,
        scratch_shapes=[pltpu.VMEM((p, n), jnp.float32)],
    )(xt, xr, centers)


def _sc_gather(table, gidx):
    d = table.shape[1]
    nidx = gidx.shape[0]
    nw = 32
    bpw = nidx // nw
    chunk = bpw
    while chunk * d * 4 > 150 * 1024:
        chunk //= 2
    nchunks = bpw // chunk
    mesh = plsc.VectorSubcoreMesh(core_axis_name="c", subcore_axis_name="s")

    @functools.partial(
        pl.kernel,
        mesh=mesh,
        out_type=jax.ShapeDtypeStruct((nidx, d), jnp.float32),
        scratch_types=[
            pltpu.VMEM((bpw,), jnp.int32),
            pltpu.VMEM((chunk, d), jnp.float32),
            pltpu.SemaphoreType.DMA,
        ],
    )
    def gather_k(table_hbm, idx_hbm, out_hbm, idx_v, rows_v, sem):
        wid = lax.axis_index("s") * 2 + lax.axis_index("c")
        base = wid * bpw
        pltpu.sync_copy(idx_hbm.at[pl.ds(base, bpw)], idx_v)

        def body(i, carry):
            off = i * chunk
            pltpu.async_copy(
                table_hbm.at[idx_v.at[pl.ds(off, chunk)]], rows_v, sem
            ).wait()
            pltpu.sync_copy(rows_v, out_hbm.at[pl.ds(base + off, chunk)])
            return carry

        lax.fori_loop(0, nchunks, body, 0)

    return gather_k(table, gidx)


def _mcg_body(gf_ref, rel_ref, dist_ref, c_ref, w1a_ref, b1a_ref, w1b_ref,
              b1b_ref, w2a_ref, b2a_ref, w2b_ref, b2b_ref, wproj_ref, out_ref,
              *, nsample):
    r = c_ref.shape[0]
    c = c_ref[...]
    c3 = jnp.broadcast_to(c[:, None, :], (r, nsample, 3)).reshape(r * nsample, 3)
    geo = jnp.concatenate([rel_ref[...], c3, dist_ref[...]], axis=1)
    h = jnp.maximum(
        jnp.dot(geo, w1a_ref[...], preferred_element_type=jnp.float32)
        + b1a_ref[...], 0.0)
    w = jnp.dot(h, w1b_ref[...], preferred_element_type=jnp.float32) + b1b_ref[...]
    z = jnp.maximum(w * w2a_ref[...] + b2a_ref[...], 0.0)
    lw = jnp.maximum(
        jnp.dot(z, w2b_ref[...], preferred_element_type=jnp.float32)
        + b2b_ref[...], 0.0)
    proj = jnp.dot(gf_ref[...], wproj_ref[...], preferred_element_type=jnp.float32)
    prod = (proj * lw).reshape(r, nsample, -1)
    out_ref[...] = jnp.max(prod, axis=1)


def _full(shape):
    return pl.BlockSpec(shape, lambda i: tuple(0 for _ in shape))


def _mcg(gf, rel, dist, centers_flat, p, cin, cout, nsample):
    rows = centers_flat.shape[0]
    r = min(64, rows)
    grid = (rows // r,)
    h1 = p['W1a'].shape[1]
    h2 = p['W2a'].shape[1]
    return pl.pallas_call(
        functools.partial(_mcg_body, nsample=nsample),
        grid=grid,
        in_specs=[
            pl.BlockSpec((r * nsample, cin), lambda i: (i, 0)),
            pl.BlockSpec((r * nsample, 3), lambda i: (i, 0)),
            pl.BlockSpec((r * nsample, 1), lambda i: (i, 0)),
            pl.BlockSpec((r, 3), lambda i: (i, 0)),
            _full((7, h1)), _full((1, h1)), _full((h1, 1)), _full((1, 1)),
            _full((1, h2)), _full((1, h2)), _full((h2, cout)), _full((1, cout)),
            _full((cin, cout)),
        ],
        out_specs=pl.BlockSpec((r, cout), lambda i: (i, 0)),
        out_shape=jax.ShapeDtypeStruct((rows, cout), jnp.float32),
    )(gf, rel, dist, centers_flat,
      p['W1a'], p['b1a'][None, :], p['W1b'], p['b1b'][None, :],
      p['W2a'], p['b2a'][None, :], p['W2b'], p['b2b'][None, :], p['Wproj'])


def _tail_body(g_ref, m_ref, w1a_ref, b1a_ref, w1b_ref, b1b_ref,
               w2a_ref, b2a_ref, w2b_ref, b2b_ref, wproj_ref,
               fc1w_ref, a1_ref, fc2w_ref, a2_ref, fc3w_ref, b3_ref, out_ref,
               *, cin, nsample, batch):
    g = g_ref[...]
    gf = g[:, :cin]
    gx = g[:, cin:cin + 3]
    c = jnp.dot(m_ref[...], gx, preferred_element_type=jnp.float32)
    c3 = jnp.broadcast_to(c[:, None, :], (batch, nsample, 3)).reshape(
        batch * nsample, 3)
    rel = gx - c3
    dist = jnp.sqrt(jnp.sum(rel * rel, axis=1, keepdims=True) + 1e-8)
    geo = jnp.concatenate([rel, c3, dist], axis=1)
    h = jnp.maximum(
        jnp.dot(geo, w1a_ref[...], preferred_element_type=jnp.float32)
        + b1a_ref[...], 0.0)
    w = jnp.dot(h, w1b_ref[...], preferred_element_type=jnp.float32) + b1b_ref[...]
    z = jnp.maximum(w * w2a_ref[...] + b2a_ref[...], 0.0)
    lw = jnp.maximum(
        jnp.dot(z, w2b_ref[...], preferred_element_type=jnp.float32)
        + b2b_ref[...], 0.0)
    proj = jnp.dot(gf, wproj_ref[...], preferred_element_type=jnp.float32)
    f = jnp.max((proj * lw).reshape(batch, nsample, -1), axis=1)
    es = 1.0 / jnp.sqrt(1.0 + 1e-5)
    x = jnp.maximum(
        jnp.dot(f, fc1w_ref[...], preferred_element_type=jnp.float32) * es
        * a1_ref[0:1, :] + a1_ref[1:2, :], 0.0)
    x = jnp.maximum(
        jnp.dot(x, fc2w_ref[...], preferred_element_type=jnp.float32) * es
        * a2_ref[0:1, :] + a2_ref[1:2, :], 0.0)
    out_ref[...] = (jnp.dot(x, fc3w_ref[...], preferred_element_type=jnp.float32)
                    + b3_ref[0:1, :])


def _tail(s4, f4, p, head):
    batch, nsample, cin = f4.shape
    gx = s4.reshape(batch * nsample, 3)
    gf = f4.reshape(batch * nsample, cin)
    pad = jnp.zeros((batch * nsample, 13), jnp.float32)
    g = jnp.concatenate([gf, gx, pad], axis=1)
    mavg = jnp.repeat(jnp.eye(batch, dtype=jnp.float32), nsample, axis=1) / nsample
    es = 1.0 / jnp.sqrt(1.0 + 1e-5)
    a1 = jnp.stack([head['bn1g'], head['fc1b'] * es * head['bn1g'] + head['bn1b']], 0)
    a2 = jnp.stack([head['bn2g'], head['fc2b'] * es * head['bn2g'] + head['bn2b']], 0)
    return pl.pallas_call(
        functools.partial(_tail_body, cin=cin, nsample=nsample, batch=batch),
        out_shape=jax.ShapeDtypeStruct((batch, 40), jnp.float32),
    )(g, mavg, p['W1a'], p['b1a'][None, :], p['W1b'], p['b1b'][None, :],
      p['W2a'], p['b2a'][None, :], p['W2b'], p['b2b'][None, :], p['Wproj'],
      head['fc1W'], a1, head['fc2W'], a2, head['fc3W'], head['fc3b'][None, :])


def _sa_layer(xt, pts, feats, p, npoint, nsample):
    batch, n, cin = feats.shape
    stride = n // npoint
    centers = pts[:, ::stride, :][:, :npoint, :]
    rows = batch * npoint
    if cin == 3:
        xr = jnp.concatenate([pts, feats], axis=2)
        _, dist, rel, gfe = _knn(xt, xr, centers, nsample)
        gf = gfe.reshape(rows * nsample, 3)
    else:
        idx, dist, rel = _knn(xt, pts, centers, nsample)
        gf = _sc_gather(feats.reshape(batch * n, cin), idx.reshape(-1))
    cout = p['W2b'].shape[1]
    f = _mcg(gf, rel.reshape(rows * nsample, 3),
             dist.reshape(rows * nsample, 1), centers.reshape(rows, 3),
             p, cin, cout, nsample)
    return f.reshape(batch, npoint, cout), centers


def kernel(xyz, params):
    ptsT = xyz[:, :3, :]
    pts = jnp.transpose(ptsT, (0, 2, 1))
    norm = jnp.transpose(xyz[:, 3:, :], (0, 2, 1))

    f1, s1 = _sa_layer(ptsT, pts, norm, params['layer0'], 512, 32)
    s1t = jnp.transpose(s1, (0, 2, 1))
    f2, s2 = _sa_layer(s1t, s1, f1, params['layer1'], 256, 32)
    s2t = jnp.transpose(s2, (0, 2, 1))
    f3, s3 = _sa_layer(s2t, s2, f2, params['layer2'], 128, 32)
    s3t = jnp.transpose(s3, (0, 2, 1))
    f4, s4 = _sa_layer(s3t, s3, f3, params['layer3'], 32, 32)

    x = _tail(s4, f4, params['layer4'], params['head'])
    return (x, s3)


# speedup vs baseline: 7.0005x; 1.1681x over previous
import functools

import jax
import jax.numpy as jnp
from jax import lax
from jax.experimental import pallas as pl
from jax.experimental.pallas import tpu as pltpu
from jax.experimental.pallas import tpu_sc as plsc


def _knn_body(xt_ref, xr_ref, c_ref, idx_ref, dist_ref, rel_ref, *rest,
              n, nsample, nf):
    d2_ref = rest[-1]
    feat_ref = rest[0] if nf > 3 else None
    b = pl.program_id(0)
    p = c_ref.shape[1]
    c = c_ref[0]
    d2 = (xt_ref[0, 0:1, :] - c[:, 0:1]) ** 2
    d2 = d2 + (xt_ref[0, 1:2, :] - c[:, 1:2]) ** 2
    d2 = d2 + (xt_ref[0, 2:3, :] - c[:, 2:3]) ** 2
    d2_ref[...] = d2
    iota = lax.broadcasted_iota(jnp.int32, (p, n), 1)
    base = b * n
    xr = xr_ref[0]
    icols, dcols, rcols, fcols = [], [], [], []
    for _ in range(nsample):
        d2 = d2_ref[...]
        m = jnp.min(d2, axis=1, keepdims=True)
        amin = jnp.min(jnp.where(d2 == m, iota, n), axis=1, keepdims=True)
        icols.append(amin + base)
        dcols.append(jnp.sqrt(m + 1e-8))
        eqf = (iota == amin).astype(jnp.float32)
        sel = jnp.dot(eqf, xr, preferred_element_type=jnp.float32)
        rcols.append(sel[:, 0:3] - c)
        if nf > 3:
            fcols.append(sel[:, 3:nf])
        d2_ref[...] = jnp.where(iota == amin, jnp.inf, d2)
    idx_ref[0] = jnp.concatenate(icols, axis=1)
    dist_ref[0] = jnp.concatenate(dcols, axis=1)
    rel_ref[0] = jnp.concatenate(rcols, axis=1)
    if nf > 3:
        feat_ref[0] = jnp.concatenate(fcols, axis=1)


def _knn(xt, xr, centers, nsample):
    batch, _, n = xt.shape
    nf = xr.shape[2]
    npoint = centers.shape[1]
    p = min(256, npoint)
    grid = (batch, npoint // p)
    out_shape = [
        jax.ShapeDtypeStruct((batch, npoint, nsample), jnp.int32),
        jax.ShapeDtypeStruct((batch, npoint, nsample), jnp.float32),
        jax.ShapeDtypeStruct((batch, npoint, nsample * 3), jnp.float32),
    ]
    out_specs = [
        pl.BlockSpec((1, p, nsample), lambda b, j: (b, j, 0)),
        pl.BlockSpec((1, p, nsample), lambda b, j: (b, j, 0)),
        pl.BlockSpec((1, p, nsample * 3), lambda b, j: (b, j, 0)),
    ]
    if nf > 3:
        out_shape.append(
            jax.ShapeDtypeStruct((batch, npoint, nsample * (nf - 3)), jnp.float32))
        out_specs.append(
            pl.BlockSpec((1, p, nsample * (nf - 3)), lambda b, j: (b, j, 0)))
    return pl.pallas_call(
        functools.partial(_knn_body, n=n, nsample=nsample, nf=nf),
        grid=grid,
        in_specs=[
            pl.BlockSpec((1, 3, n), lambda b, j: (b, 0, 0)),
            pl.BlockSpec((1, n, nf), lambda b, j: (b, 0, 0)),
            pl.BlockSpec((1, p, 3), lambda b, j: (b, j, 0)),
        ],
        out_specs=out_specs,
        out_shape=out_shape,
        scratch_shapes=[pltpu.VMEM((p, n), jnp.float32)],
    )(xt, xr, centers)


def _sc_gather(table, gidx):
    d = table.shape[1]
    nidx = gidx.shape[0]
    nw = 32
    bpw = nidx // nw
    chunk = bpw
    while chunk * d * 4 > 150 * 1024:
        chunk //= 2
    nchunks = bpw // chunk
    mesh = plsc.VectorSubcoreMesh(core_axis_name="c", subcore_axis_name="s")

    @functools.partial(
        pl.kernel,
        mesh=mesh,
        out_type=jax.ShapeDtypeStruct((nidx, d), jnp.float32),
        scratch_types=[
            pltpu.VMEM((bpw,), jnp.int32),
            pltpu.VMEM((chunk, d), jnp.float32),
            pltpu.SemaphoreType.DMA,
        ],
    )
    def gather_k(table_hbm, idx_hbm, out_hbm, idx_v, rows_v, sem):
        wid = lax.axis_index("s") * 2 + lax.axis_index("c")
        base = wid * bpw
        pltpu.sync_copy(idx_hbm.at[pl.ds(base, bpw)], idx_v)

        def body(i, carry):
            off = i * chunk
            pltpu.async_copy(
                table_hbm.at[idx_v.at[pl.ds(off, chunk)]], rows_v, sem
            ).wait()
            pltpu.sync_copy(rows_v, out_hbm.at[pl.ds(base + off, chunk)])
            return carry

        lax.fori_loop(0, nchunks, body, 0)

    return gather_k(table, gidx)


def _mcg_body(gf_ref, rel_ref, dist_ref, c_ref, w1a_ref, b1a_ref, w1b_ref,
              b1b_ref, w2a_ref, b2a_ref, w2b_ref, b2b_ref, wproj_ref, out_ref,
              *, nsample):
    r = c_ref.shape[0]
    c = c_ref[...]
    c3 = jnp.broadcast_to(c[:, None, :], (r, nsample, 3)).reshape(r * nsample, 3)
    geo = jnp.concatenate([rel_ref[...], c3, dist_ref[...]], axis=1)
    h = jnp.maximum(
        jnp.dot(geo, w1a_ref[...], preferred_element_type=jnp.float32)
        + b1a_ref[...], 0.0)
    w = jnp.dot(h, w1b_ref[...], preferred_element_type=jnp.float32) + b1b_ref[...]
    z = jnp.maximum(w * w2a_ref[...] + b2a_ref[...], 0.0)
    lw = jnp.maximum(
        jnp.dot(z, w2b_ref[...], preferred_element_type=jnp.float32)
        + b2b_ref[...], 0.0)
    proj = jnp.dot(gf_ref[...], wproj_ref[...], preferred_element_type=jnp.float32)
    prod = (proj * lw).reshape(r, nsample, -1)
    out_ref[...] = jnp.max(prod, axis=1)


def _full(shape):
    return pl.BlockSpec(shape, lambda i: tuple(0 for _ in shape))


def _mcg(gf, rel, dist, centers_flat, p, cin, cout, nsample):
    rows = centers_flat.shape[0]
    r = min(64, rows)
    grid = (rows // r,)
    h1 = p['W1a'].shape[1]
    h2 = p['W2a'].shape[1]
    return pl.pallas_call(
        functools.partial(_mcg_body, nsample=nsample),
        grid=grid,
        in_specs=[
            pl.BlockSpec((r * nsample, cin), lambda i: (i, 0)),
            pl.BlockSpec((r * nsample, 3), lambda i: (i, 0)),
            pl.BlockSpec((r * nsample, 1), lambda i: (i, 0)),
            pl.BlockSpec((r, 3), lambda i: (i, 0)),
            _full((7, h1)), _full((1, h1)), _full((h1, 1)), _full((1, 1)),
            _full((1, h2)), _full((1, h2)), _full((h2, cout)), _full((1, cout)),
            _full((cin, cout)),
        ],
        out_specs=pl.BlockSpec((r, cout), lambda i: (i, 0)),
        out_shape=jax.ShapeDtypeStruct((rows, cout), jnp.float32),
    )(gf, rel, dist, centers_flat,
      p['W1a'], p['b1a'][None, :], p['W1b'], p['b1b'][None, :],
      p['W2a'], p['b2a'][None, :], p['W2b'], p['b2b'][None, :], p['Wproj'])


def _tail_body(g_ref, m_ref, w1a_ref, b1a_ref, w1b_ref, b1b_ref,
               w2a_ref, b2a_ref, w2b_ref, b2b_ref, wproj_ref,
               fc1w_ref, a1_ref, fc2w_ref, a2_ref, fc3w_ref, b3_ref, out_ref,
               *, cin, nsample, batch):
    g = g_ref[...]
    gf = g[:, :cin]
    gx = g[:, cin:cin + 3]
    c = jnp.dot(m_ref[...], gx, preferred_element_type=jnp.float32)
    c3 = jnp.broadcast_to(c[:, None, :], (batch, nsample, 3)).reshape(
        batch * nsample, 3)
    rel = gx - c3
    dist = jnp.sqrt(jnp.sum(rel * rel, axis=1, keepdims=True) + 1e-8)
    geo = jnp.concatenate([rel, c3, dist], axis=1)
    h = jnp.maximum(
        jnp.dot(geo, w1a_ref[...], preferred_element_type=jnp.float32)
        + b1a_ref[...], 0.0)
    w = jnp.dot(h, w1b_ref[...], preferred_element_type=jnp.float32) + b1b_ref[...]
    z = jnp.maximum(w * w2a_ref[...] + b2a_ref[...], 0.0)
    lw = jnp.maximum(
        jnp.dot(z, w2b_ref[...], preferred_element_type=jnp.float32)
        + b2b_ref[...], 0.0)
    proj = jnp.dot(gf, wproj_ref[...], preferred_element_type=jnp.float32)
    f = jnp.max((proj * lw).reshape(batch, nsample, -1), axis=1)
    es = 1.0 / jnp.sqrt(1.0 + 1e-5)
    x = jnp.maximum(
        jnp.dot(f, fc1w_ref[...], preferred_element_type=jnp.float32) * es
        * a1_ref[0:1, :] + a1_ref[1:2, :], 0.0)
    x = jnp.maximum(
        jnp.dot(x, fc2w_ref[...], preferred_element_type=jnp.float32) * es
        * a2_ref[0:1, :] + a2_ref[1:2, :], 0.0)
    out_ref[...] = (jnp.dot(x, fc3w_ref[...], preferred_element_type=jnp.float32)
                    + b3_ref[0:1, :])


def _tail(s4, f4, p, head):
    batch, nsample, cin = f4.shape
    gx = s4.reshape(batch * nsample, 3)
    gf = f4.reshape(batch * nsample, cin)
    pad = jnp.zeros((batch * nsample, 13), jnp.float32)
    g = jnp.concatenate([gf, gx, pad], axis=1)
    mavg = jnp.repeat(jnp.eye(batch, dtype=jnp.float32), nsample, axis=1) / nsample
    es = 1.0 / jnp.sqrt(1.0 + 1e-5)
    a1 = jnp.stack([head['bn1g'], head['fc1b'] * es * head['bn1g'] + head['bn1b']], 0)
    a2 = jnp.stack([head['bn2g'], head['fc2b'] * es * head['bn2g'] + head['bn2b']], 0)
    return pl.pallas_call(
        functools.partial(_tail_body, cin=cin, nsample=nsample, batch=batch),
        out_shape=jax.ShapeDtypeStruct((batch, 40), jnp.float32),
    )(g, mavg, p['W1a'], p['b1a'][None, :], p['W1b'], p['b1b'][None, :],
      p['W2a'], p['b2a'][None, :], p['W2b'], p['b2b'][None, :], p['Wproj'],
      head['fc1W'], a1, head['fc2W'], a2, head['fc3W'], head['fc3b'][None, :])


def _sa_layer(xt, pts, feats, p, npoint, nsample):
    batch, n, cin = feats.shape
    stride = n // npoint
    centers = pts[:, ::stride, :][:, :npoint, :]
    rows = batch * npoint
    if cin == 3:
        xr = jnp.concatenate([pts, feats], axis=2)
        _, dist, rel, gfe = _knn(xt, xr, centers, nsample)
        gf = gfe.reshape(rows * nsample, 3)
    else:
        idx, dist, rel = _knn(xt, pts, centers, nsample)
        gf = _sc_gather(feats.reshape(batch * n, cin), idx.reshape(-1))
    cout = p['W2b'].shape[1]
    f = _mcg(gf, rel.reshape(rows * nsample, 3),
             dist.reshape(rows * nsample, 1), centers.reshape(rows, 3),
             p, cin, cout, nsample)
    return f.reshape(batch, npoint, cout), centers


def kernel(xyz, params):
    ptsT = xyz[:, :3, :]
    pts = jnp.transpose(ptsT, (0, 2, 1))
    norm = jnp.transpose(xyz[:, 3:, :], (0, 2, 1))

    f1, s1 = _sa_layer(ptsT, pts, norm, params['layer0'], 512, 32)
    s1t = jnp.transpose(s1, (0, 2, 1))
    f2, s2 = _sa_layer(s1t, s1, f1, params['layer1'], 256, 32)
    s2t = jnp.transpose(s2, (0, 2, 1))
    f3, s3 = _sa_layer(s2t, s2, f2, params['layer2'], 128, 32)
    s3t = jnp.transpose(s3, (0, 2, 1))
    f4, s4 = _sa_layer(s3t, s3, f3, params['layer3'], 32, 32)

    x = _tail(s4, f4, params['layer4'], params['head'])
    return (x, s3)


# speedup vs baseline: 11.3078x; 1.6153x over previous
import functools

import jax
import jax.numpy as jnp
from jax import lax
from jax.experimental import pallas as pl
from jax.experimental.pallas import tpu as pltpu
from jax.experimental.pallas import tpu_sc as plsc


def _knn_body(xt_ref, c_ref, idx_ref, d2_ref, *, n, nsample):
    b = pl.program_id(0)
    p = c_ref.shape[1]
    c = c_ref[0]
    d2 = (xt_ref[0, 0:1, :] - c[:, 0:1]) ** 2
    d2 = d2 + (xt_ref[0, 1:2, :] - c[:, 1:2]) ** 2
    d2 = d2 + (xt_ref[0, 2:3, :] - c[:, 2:3]) ** 2
    d2_ref[...] = d2
    iota = lax.broadcasted_iota(jnp.int32, (p, n), 1)
    base = b * n
    icols = []
    for _ in range(nsample):
        d2 = d2_ref[...]
        m = jnp.min(d2, axis=1, keepdims=True)
        amin = jnp.min(jnp.where(d2 == m, iota, n), axis=1, keepdims=True)
        icols.append(amin + base)
        d2_ref[...] = jnp.where(iota == amin, jnp.inf, d2)
    idx_ref[0] = jnp.concatenate(icols, axis=1)


def _knn(xt, centers, nsample):
    batch, _, n = xt.shape
    npoint = centers.shape[1]
    p = min(256, npoint)
    grid = (batch, npoint // p)
    return pl.pallas_call(
        functools.partial(_knn_body, n=n, nsample=nsample),
        grid=grid,
        in_specs=[
            pl.BlockSpec((1, 3, n), lambda b, j: (b, 0, 0)),
            pl.BlockSpec((1, p, 3), lambda b, j: (b, j, 0)),
        ],
        out_specs=pl.BlockSpec((1, p, nsample), lambda b, j: (b, j, 0)),
        out_shape=jax.ShapeDtypeStruct((batch, npoint, nsample), jnp.int32),
        scratch_shapes=[pltpu.VMEM((p, n), jnp.float32)],
    )(xt, centers)


def _sc_gather(table, gidx):
    d = table.shape[1]
    nidx = gidx.shape[0]
    nw = 32
    bpw = nidx // nw
    chunk = bpw
    while chunk * d * 4 > 150 * 1024:
        chunk //= 2
    nchunks = bpw // chunk
    mesh = plsc.VectorSubcoreMesh(core_axis_name="c", subcore_axis_name="s")

    @functools.partial(
        pl.kernel,
        mesh=mesh,
        out_type=jax.ShapeDtypeStruct((nidx, d), jnp.float32),
        scratch_types=[
            pltpu.VMEM((bpw,), jnp.int32),
            pltpu.VMEM((chunk, d), jnp.float32),
            pltpu.SemaphoreType.DMA,
        ],
    )
    def gather_k(table_hbm, idx_hbm, out_hbm, idx_v, rows_v, sem):
        wid = lax.axis_index("s") * 2 + lax.axis_index("c")
        base = wid * bpw
        pltpu.sync_copy(idx_hbm.at[pl.ds(base, bpw)], idx_v)

        def body(i, carry):
            off = i * chunk
            pltpu.async_copy(
                table_hbm.at[idx_v.at[pl.ds(off, chunk)]], rows_v, sem
            ).wait()
            pltpu.sync_copy(rows_v, out_hbm.at[pl.ds(base + off, chunk)])
            return carry

        lax.fori_loop(0, nchunks, body, 0)

    return gather_k(table, gidx)


def _mcg_body(g_ref, c_ref, w1a_ref, b1a_ref, w1b_ref,
              b1b_ref, w2a_ref, b2a_ref, w2b_ref, b2b_ref, wproj_ref, out_ref,
              *, cin, nsample):
    r = c_ref.shape[0]
    g = g_ref[...]
    gx = g[:, 0:3]
    gf = g[:, 3:3 + cin]
    c = c_ref[...]
    c3 = jnp.broadcast_to(c[:, None, :], (r, nsample, 3)).reshape(r * nsample, 3)
    rel = gx - c3
    dist = jnp.sqrt(jnp.sum(rel * rel, axis=1, keepdims=True) + 1e-8)
    geo = jnp.concatenate([rel, c3, dist], axis=1)
    h = jnp.maximum(
        jnp.dot(geo, w1a_ref[...], preferred_element_type=jnp.float32)
        + b1a_ref[...], 0.0)
    w = jnp.dot(h, w1b_ref[...], preferred_element_type=jnp.float32) + b1b_ref[...]
    z = jnp.maximum(w * w2a_ref[...] + b2a_ref[...], 0.0)
    lw = jnp.maximum(
        jnp.dot(z, w2b_ref[...], preferred_element_type=jnp.float32)
        + b2b_ref[...], 0.0)
    proj = jnp.dot(gf, wproj_ref[...], preferred_element_type=jnp.float32)
    prod = (proj * lw).reshape(r, nsample, -1)
    out_ref[...] = jnp.max(prod, axis=1)


def _full(shape):
    return pl.BlockSpec(shape, lambda i: tuple(0 for _ in shape))


def _mcg(g, centers_flat, p, cin, cout, nsample):
    rows = centers_flat.shape[0]
    d = g.shape[1]
    r = min(64, rows)
    grid = (rows // r,)
    h1 = p['W1a'].shape[1]
    h2 = p['W2a'].shape[1]
    return pl.pallas_call(
        functools.partial(_mcg_body, cin=cin, nsample=nsample),
        grid=grid,
        in_specs=[
            pl.BlockSpec((r * nsample, d), lambda i: (i, 0)),
            pl.BlockSpec((r, 3), lambda i: (i, 0)),
            _full((7, h1)), _full((1, h1)), _full((h1, 1)), _full((1, 1)),
            _full((1, h2)), _full((1, h2)), _full((h2, cout)), _full((1, cout)),
            _full((cin, cout)),
        ],
        out_specs=pl.BlockSpec((r, cout), lambda i: (i, 0)),
        out_shape=jax.ShapeDtypeStruct((rows, cout), jnp.float32),
    )(g, centers_flat,
      p['W1a'], p['b1a'][None, :], p['W1b'], p['b1b'][None, :],
      p['W2a'], p['b2a'][None, :], p['W2b'], p['b2b'][None, :], p['Wproj'])


def _tail_body(g_ref, m_ref, w1a_ref, b1a_ref, w1b_ref, b1b_ref,
               w2a_ref, b2a_ref, w2b_ref, b2b_ref, wproj_ref,
               fc1w_ref, a1_ref, fc2w_ref, a2_ref, fc3w_ref, b3_ref, out_ref,
               *, cin, nsample, batch):
    g = g_ref[...]
    gf = g[:, :cin]
    gx = g[:, cin:cin + 3]
    c = jnp.dot(m_ref[...], gx, preferred_element_type=jnp.float32)
    c3 = jnp.broadcast_to(c[:, None, :], (batch, nsample, 3)).reshape(
        batch * nsample, 3)
    rel = gx - c3
    dist = jnp.sqrt(jnp.sum(rel * rel, axis=1, keepdims=True) + 1e-8)
    geo = jnp.concatenate([rel, c3, dist], axis=1)
    h = jnp.maximum(
        jnp.dot(geo, w1a_ref[...], preferred_element_type=jnp.float32)
        + b1a_ref[...], 0.0)
    w = jnp.dot(h, w1b_ref[...], preferred_element_type=jnp.float32) + b1b_ref[...]
    z = jnp.maximum(w * w2a_ref[...] + b2a_ref[...], 0.0)
    lw = jnp.maximum(
        jnp.dot(z, w2b_ref[...], preferred_element_type=jnp.float32)
        + b2b_ref[...], 0.0)
    proj = jnp.dot(gf, wproj_ref[...], preferred_element_type=jnp.float32)
    f = jnp.max((proj * lw).reshape(batch, nsample, -1), axis=1)
    es = 1.0 / jnp.sqrt(1.0 + 1e-5)
    x = jnp.maximum(
        jnp.dot(f, fc1w_ref[...], preferred_element_type=jnp.float32) * es
        * a1_ref[0:1, :] + a1_ref[1:2, :], 0.0)
    x = jnp.maximum(
        jnp.dot(x, fc2w_ref[...], preferred_element_type=jnp.float32) * es
        * a2_ref[0:1, :] + a2_ref[1:2, :], 0.0)
    out_ref[...] = (jnp.dot(x, fc3w_ref[...], preferred_element_type=jnp.float32)
                    + b3_ref[0:1, :])


def _tail(s4, f4, p, head):
    batch, nsample, cin = f4.shape
    gx = s4.reshape(batch * nsample, 3)
    gf = f4.reshape(batch * nsample, cin)
    pad = jnp.zeros((batch * nsample, 13), jnp.float32)
    g = jnp.concatenate([gf, gx, pad], axis=1)
    mavg = jnp.repeat(jnp.eye(batch, dtype=jnp.float32), nsample, axis=1) / nsample
    es = 1.0 / jnp.sqrt(1.0 + 1e-5)
    a1 = jnp.stack([head['bn1g'], head['fc1b'] * es * head['bn1g'] + head['bn1b']], 0)
    a2 = jnp.stack([head['bn2g'], head['fc2b'] * es * head['bn2g'] + head['bn2b']], 0)
    return pl.pallas_call(
        functools.partial(_tail_body, cin=cin, nsample=nsample, batch=batch),
        out_shape=jax.ShapeDtypeStruct((batch, 40), jnp.float32),
    )(g, mavg, p['W1a'], p['b1a'][None, :], p['W1b'], p['b1b'][None, :],
      p['W2a'], p['b2a'][None, :], p['W2b'], p['b2b'][None, :], p['Wproj'],
      head['fc1W'], a1, head['fc2W'], a2, head['fc3W'], head['fc3b'][None, :])


def _sa_layer(xt, pts, feats, p, npoint, nsample):
    batch, n, cin = feats.shape
    stride = n // npoint
    centers = pts[:, ::stride, :][:, :npoint, :]
    rows = batch * npoint
    idx = _knn(xt, centers, nsample)
    d = 3 + cin
    dpad = (d + 127) // 128 * 128
    table = jnp.concatenate(
        [pts, feats, jnp.zeros((batch, n, dpad - d), jnp.float32)], axis=2
    ).reshape(batch * n, dpad)
    g = _sc_gather(table, idx.reshape(-1))
    cout = p['W2b'].shape[1]
    f = _mcg(g, centers.reshape(rows, 3), p, cin, cout, nsample)
    return f.reshape(batch, npoint, cout), centers


def kernel(xyz, params):
    ptsT = xyz[:, :3, :]
    pts = jnp.transpose(ptsT, (0, 2, 1))
    norm = jnp.transpose(xyz[:, 3:, :], (0, 2, 1))

    f1, s1 = _sa_layer(ptsT, pts, norm, params['layer0'], 512, 32)
    s1t = jnp.transpose(s1, (0, 2, 1))
    f2, s2 = _sa_layer(s1t, s1, f1, params['layer1'], 256, 32)
    s2t = jnp.transpose(s2, (0, 2, 1))
    f3, s3 = _sa_layer(s2t, s2, f2, params['layer2'], 128, 32)
    s3t = jnp.transpose(s3, (0, 2, 1))
    f4, s4 = _sa_layer(s3t, s3, f3, params['layer3'], 32, 32)

    x = _tail(s4, f4, params['layer4'], params['head'])
    return (x, s3)


# speedup vs baseline: 11.3161x; 1.0007x over previous
import functools

import jax
import jax.numpy as jnp
from jax import lax
from jax.experimental import pallas as pl
from jax.experimental.pallas import tpu as pltpu
from jax.experimental.pallas import tpu_sc as plsc


def _knn_body(xt_ref, c_ref, idx_ref, d2_ref, *, n, nsample):
    b = pl.program_id(0)
    p = c_ref.shape[1]
    c = c_ref[0]
    d2 = (xt_ref[0, 0:1, :] - c[:, 0:1]) ** 2
    d2 = d2 + (xt_ref[0, 1:2, :] - c[:, 1:2]) ** 2
    d2 = d2 + (xt_ref[0, 2:3, :] - c[:, 2:3]) ** 2
    d2_ref[...] = d2
    iota = lax.broadcasted_iota(jnp.int32, (p, n), 1)
    base = b * n
    icols = []
    for _ in range(nsample):
        d2 = d2_ref[...]
        m = jnp.min(d2, axis=1, keepdims=True)
        amin = jnp.min(jnp.where(d2 == m, iota, n), axis=1, keepdims=True)
        icols.append(amin + base)
        d2_ref[...] = jnp.where(iota == amin, jnp.inf, d2)
    idx_ref[0] = jnp.concatenate(icols, axis=1)


def _knn(xt, centers, nsample):
    batch, _, n = xt.shape
    npoint = centers.shape[1]
    p = min(256, npoint)
    grid = (batch, npoint // p)
    return pl.pallas_call(
        functools.partial(_knn_body, n=n, nsample=nsample),
        grid=grid,
        in_specs=[
            pl.BlockSpec((1, 3, n), lambda b, j: (b, 0, 0)),
            pl.BlockSpec((1, p, 3), lambda b, j: (b, j, 0)),
        ],
        out_specs=pl.BlockSpec((1, p, nsample), lambda b, j: (b, j, 0)),
        out_shape=jax.ShapeDtypeStruct((batch, npoint, nsample), jnp.int32),
        scratch_shapes=[pltpu.VMEM((p, n), jnp.float32)],
    )(xt, centers)


def _sc_gather(table, gidx):
    d = table.shape[1]
    nidx = gidx.shape[0]
    nw = 32
    bpw = nidx // nw
    chunk = bpw
    while chunk * d * 4 > 400 * 1024:
        chunk //= 2
    nchunks = bpw // chunk
    mesh = plsc.VectorSubcoreMesh(core_axis_name="c", subcore_axis_name="s")

    @functools.partial(
        pl.kernel,
        mesh=mesh,
        out_type=jax.ShapeDtypeStruct((nidx, d), jnp.float32),
        scratch_types=[
            pltpu.VMEM((bpw,), jnp.int32),
            pltpu.VMEM((chunk, d), jnp.float32),
            pltpu.SemaphoreType.DMA,
        ],
    )
    def gather_k(table_hbm, idx_hbm, out_hbm, idx_v, rows_v, sem):
        wid = lax.axis_index("s") * 2 + lax.axis_index("c")
        base = wid * bpw
        pltpu.sync_copy(idx_hbm.at[pl.ds(base, bpw)], idx_v)

        def body(i, carry):
            off = i * chunk
            pltpu.async_copy(
                table_hbm.at[idx_v.at[pl.ds(off, chunk)]], rows_v, sem
            ).wait()
            pltpu.sync_copy(rows_v, out_hbm.at[pl.ds(base + off, chunk)])
            return carry

        lax.fori_loop(0, nchunks, body, 0)

    return gather_k(table, gidx)


def _mcg_body(g_ref, c_ref, w1a_ref, b1a_ref, w1b_ref,
              b1b_ref, w2a_ref, b2a_ref, w2b_ref, b2b_ref, wproj_ref, out_ref,
              *, cin, nsample):
    r = c_ref.shape[0]
    g = g_ref[...]
    gx = g[:, 0:3]
    gf = g[:, 3:3 + cin]
    c = c_ref[...]
    c3 = jnp.broadcast_to(c[:, None, :], (r, nsample, 3)).reshape(r * nsample, 3)
    rel = gx - c3
    dist = jnp.sqrt(jnp.sum(rel * rel, axis=1, keepdims=True) + 1e-8)
    geo = jnp.concatenate([rel, c3, dist], axis=1)
    h = jnp.maximum(
        jnp.dot(geo, w1a_ref[...], preferred_element_type=jnp.float32)
        + b1a_ref[...], 0.0)
    w = jnp.dot(h, w1b_ref[...], preferred_element_type=jnp.float32) + b1b_ref[...]
    z = jnp.maximum(w * w2a_ref[...] + b2a_ref[...], 0.0)
    lw = jnp.maximum(
        jnp.dot(z, w2b_ref[...], preferred_element_type=jnp.float32)
        + b2b_ref[...], 0.0)
    proj = jnp.dot(gf, wproj_ref[...], preferred_element_type=jnp.float32)
    prod = (proj * lw).reshape(r, nsample, -1)
    out_ref[...] = jnp.max(prod, axis=1)


def _full(shape):
    return pl.BlockSpec(shape, lambda i: tuple(0 for _ in shape))


def _mcg(g, centers_flat, p, cin, cout, nsample):
    rows = centers_flat.shape[0]
    d = g.shape[1]
    r = min(64, rows)
    grid = (rows // r,)
    h1 = p['W1a'].shape[1]
    h2 = p['W2a'].shape[1]
    return pl.pallas_call(
        functools.partial(_mcg_body, cin=cin, nsample=nsample),
        grid=grid,
        in_specs=[
            pl.BlockSpec((r * nsample, d), lambda i: (i, 0)),
            pl.BlockSpec((r, 3), lambda i: (i, 0)),
            _full((7, h1)), _full((1, h1)), _full((h1, 1)), _full((1, 1)),
            _full((1, h2)), _full((1, h2)), _full((h2, cout)), _full((1, cout)),
            _full((cin, cout)),
        ],
        out_specs=pl.BlockSpec((r, cout), lambda i: (i, 0)),
        out_shape=jax.ShapeDtypeStruct((rows, cout), jnp.float32),
    )(g, centers_flat,
      p['W1a'], p['b1a'][None, :], p['W1b'], p['b1b'][None, :],
      p['W2a'], p['b2a'][None, :], p['W2b'], p['b2b'][None, :], p['Wproj'])


def _tail_body(g_ref, m_ref, w1a_ref, b1a_ref, w1b_ref, b1b_ref,
               w2a_ref, b2a_ref, w2b_ref, b2b_ref, wproj_ref,
               fc1w_ref, a1_ref, fc2w_ref, a2_ref, fc3w_ref, b3_ref, out_ref,
               *, cin, nsample, batch):
    g = g_ref[...]
    gf = g[:, :cin]
    gx = g[:, cin:cin + 3]
    c = jnp.dot(m_ref[...], gx, preferred_element_type=jnp.float32)
    c3 = jnp.broadcast_to(c[:, None, :], (batch, nsample, 3)).reshape(
        batch * nsample, 3)
    rel = gx - c3
    dist = jnp.sqrt(jnp.sum(rel * rel, axis=1, keepdims=True) + 1e-8)
    geo = jnp.concatenate([rel, c3, dist], axis=1)
    h = jnp.maximum(
        jnp.dot(geo, w1a_ref[...], preferred_element_type=jnp.float32)
        + b1a_ref[...], 0.0)
    w = jnp.dot(h, w1b_ref[...], preferred_element_type=jnp.float32) + b1b_ref[...]
    z = jnp.maximum(w * w2a_ref[...] + b2a_ref[...], 0.0)
    lw = jnp.maximum(
        jnp.dot(z, w2b_ref[...], preferred_element_type=jnp.float32)
        + b2b_ref[...], 0.0)
    proj = jnp.dot(gf, wproj_ref[...], preferred_element_type=jnp.float32)
    f = jnp.max((proj * lw).reshape(batch, nsample, -1), axis=1)
    es = 1.0 / jnp.sqrt(1.0 + 1e-5)
    x = jnp.maximum(
        jnp.dot(f, fc1w_ref[...], preferred_element_type=jnp.float32) * es
        * a1_ref[0:1, :] + a1_ref[1:2, :], 0.0)
    x = jnp.maximum(
        jnp.dot(x, fc2w_ref[...], preferred_element_type=jnp.float32) * es
        * a2_ref[0:1, :] + a2_ref[1:2, :], 0.0)
    out_ref[...] = (jnp.dot(x, fc3w_ref[...], preferred_element_type=jnp.float32)
                    + b3_ref[0:1, :])


def _tail(s4, f4, p, head):
    batch, nsample, cin = f4.shape
    gx = s4.reshape(batch * nsample, 3)
    gf = f4.reshape(batch * nsample, cin)
    pad = jnp.zeros((batch * nsample, 13), jnp.float32)
    g = jnp.concatenate([gf, gx, pad], axis=1)
    mavg = jnp.repeat(jnp.eye(batch, dtype=jnp.float32), nsample, axis=1) / nsample
    es = 1.0 / jnp.sqrt(1.0 + 1e-5)
    a1 = jnp.stack([head['bn1g'], head['fc1b'] * es * head['bn1g'] + head['bn1b']], 0)
    a2 = jnp.stack([head['bn2g'], head['fc2b'] * es * head['bn2g'] + head['bn2b']], 0)
    return pl.pallas_call(
        functools.partial(_tail_body, cin=cin, nsample=nsample, batch=batch),
        out_shape=jax.ShapeDtypeStruct((batch, 40), jnp.float32),
    )(g, mavg, p['W1a'], p['b1a'][None, :], p['W1b'], p['b1b'][None, :],
      p['W2a'], p['b2a'][None, :], p['W2b'], p['b2b'][None, :], p['Wproj'],
      head['fc1W'], a1, head['fc2W'], a2, head['fc3W'], head['fc3b'][None, :])


def _sa_layer(xt, pts, feats, p, npoint, nsample):
    batch, n, cin = feats.shape
    stride = n // npoint
    centers = pts[:, ::stride, :][:, :npoint, :]
    rows = batch * npoint
    idx = _knn(xt, centers, nsample)
    d = 3 + cin
    dpad = (d + 127) // 128 * 128
    table = jnp.concatenate(
        [pts, feats, jnp.zeros((batch, n, dpad - d), jnp.float32)], axis=2
    ).reshape(batch * n, dpad)
    g = _sc_gather(table, idx.reshape(-1))
    cout = p['W2b'].shape[1]
    f = _mcg(g, centers.reshape(rows, 3), p, cin, cout, nsample)
    return f.reshape(batch, npoint, cout), centers


def kernel(xyz, params):
    ptsT = xyz[:, :3, :]
    pts = jnp.transpose(ptsT, (0, 2, 1))
    norm = jnp.transpose(xyz[:, 3:, :], (0, 2, 1))

    f1, s1 = _sa_layer(ptsT, pts, norm, params['layer0'], 512, 32)
    s1t = jnp.transpose(s1, (0, 2, 1))
    f2, s2 = _sa_layer(s1t, s1, f1, params['layer1'], 256, 32)
    s2t = jnp.transpose(s2, (0, 2, 1))
    f3, s3 = _sa_layer(s2t, s2, f2, params['layer2'], 128, 32)
    s3t = jnp.transpose(s3, (0, 2, 1))
    f4, s4 = _sa_layer(s3t, s3, f3, params['layer3'], 32, 32)

    x = _tail(s4, f4, params['layer4'], params['head'])
    return (x, s3)
